# baseline (speedup 1.0000x reference)
import functools

import jax
import jax.numpy as jnp
from jax.experimental import pallas as pl
from jax.experimental.pallas import tpu as pltpu
from jax.experimental.pallas import tpu_sc as plsc

_BB = 2


def _sc_gather(table_hbm, ids_hbm, out_hbm, ids_v, rows_v, sem):
    wid = (jax.lax.axis_index("c") * jax.lax.psum(1, "s")
           + jax.lax.axis_index("s"))

    @pl.when(wid == 0)
    def _():
        pltpu.sync_copy(ids_hbm, ids_v)
        pltpu.async_copy(table_hbm.at[ids_v], rows_v, sem).wait()
        pltpu.sync_copy(rows_v, out_hbm)


def _embed_lookup(table, ids):
    b = ids.shape[0]
    ch = table.shape[1]
    k = functools.partial(
        pl.kernel,
        out_type=jax.ShapeDtypeStruct((b, ch), jnp.float32),
        mesh=plsc.VectorSubcoreMesh(core_axis_name="c", subcore_axis_name="s"),
        scratch_types=[
            pltpu.VMEM((b,), jnp.int32),
            pltpu.VMEM((b, ch), jnp.float32),
            pltpu.SemaphoreType.DMA,
        ],
    )(_sc_gather)
    return k(table, ids)


def _film_kernel(emb_ref, feat_ref, ws_ref, bs_ref, wb_ref, bb_ref, out_ref):
    ch = ws_ref.shape[1]
    for j in range(_BB):
        emb = emb_ref[0, pl.ds(j, 1), :]
        s = jnp.tanh(jnp.dot(emb, ws_ref[...],
                             preferred_element_type=jnp.float32) + bs_ref[...])
        b = jnp.tanh(jnp.dot(emb, wb_ref[...],
                             preferred_element_type=jnp.float32) + bb_ref[...])
        s4 = (1.0 + s).reshape(1, ch, 1, 1)
        b4 = b.reshape(1, ch, 1, 1)
        out_ref[pl.ds(j, 1)] = feat_ref[pl.ds(j, 1)] * s4 + b4


def kernel(feat, vessel_ids, embed_table, Ws, bs, Wb, bb):
    B, CH, H, W = feat.shape
    ng = B // _BB

    ids = vessel_ids.astype(jnp.int32)
    chp = 128
    table_p = jnp.pad(embed_table, ((0, 0), (0, chp - CH)))
    ws_p = jnp.pad(Ws, ((0, chp - CH), (0, 0)))
    wb_p = jnp.pad(Wb, ((0, chp - CH), (0, 0)))
    emb = _embed_lookup(table_p, ids)
    emb3 = emb.reshape(ng, _BB, chp)
    bs_row = bs[None, :]
    bb_row = bb[None, :]

    return pl.pallas_call(
        _film_kernel,
        grid=(ng,),
        in_specs=[
            pl.BlockSpec((1, _BB, chp), lambda g: (g, 0, 0)),
            pl.BlockSpec((_BB, CH, H, W), lambda g: (g, 0, 0, 0)),
            pl.BlockSpec((chp, CH), lambda g: (0, 0)),
            pl.BlockSpec((1, CH), lambda g: (0, 0)),
            pl.BlockSpec((chp, CH), lambda g: (0, 0)),
            pl.BlockSpec((1, CH), lambda g: (0, 0)),
        ],
        out_specs=pl.BlockSpec((_BB, CH, H, W), lambda g: (g, 0, 0, 0)),
        out_shape=jax.ShapeDtypeStruct((B, CH, H, W), jnp.float32),
    )(emb3, feat, ws_p, bs_row, wb_p, bb_row)
